# baseline (speedup 1.0000x reference)
import functools

import jax
import jax.numpy as jnp
from jax import lax
from jax.experimental import pallas as pl
from jax.experimental.pallas import tpu as pltpu
from jax.experimental.pallas import tpu_sc as plsc

NUM_ELEMENTS = 100
EMBED_DIM = 128

_B = 16384 * 200
_NC = 2
_NS = 16
_NW = _NC * _NS
_BPW = _B // _NW
_K = 8
_SUB = 128
_SUBW = _SUB * EMBED_DIM
_NBUF = 4
_NGROUP = _BPW // (_K * _SUB)
_IDX_ROWS_PER_W = _BPW // 128
_NSUBW = _BPW // _SUB
_TW = NUM_ELEMENTS * EMBED_DIM


def _make_sc_kernel():
    mesh = plsc.VectorSubcoreMesh(core_axis_name="c", subcore_axis_name="s")

    @functools.partial(
        pl.kernel,
        mesh=mesh,
        out_type=jax.ShapeDtypeStruct((_B * EMBED_DIM,), jnp.float32),
        compiler_params=pltpu.CompilerParams(needs_layout_passes=False),
        scratch_types=[
            pltpu.VMEM((2, _K, 128), jnp.int32),
            pltpu.VMEM((_NBUF, _SUBW), jnp.float32),
            pltpu.VMEM((_TW,), jnp.float32),
            pltpu.VMEM_SHARED((_TW,), jnp.float32),
            pltpu.SemaphoreType.DMA((2,)),
            pltpu.SemaphoreType.DMA((_NBUF,)),
        ],
    )
    def emb(table_hbm, idx_hbm, out_hbm, idx_v, bufs, table_tl, table_sp,
            sem_i, sem_o):
        sid = lax.axis_index("s")
        wid = sid * _NC + lax.axis_index("c")
        idx_row_base = wid * _IDX_ROWS_PER_W
        out_base = wid * _BPW * EMBED_DIM

        @pl.when(sid == 0)
        def _():
            pltpu.sync_copy(table_hbm, table_sp)

        plsc.subcore_barrier()
        pltpu.sync_copy(table_sp, table_tl)

        def fire_idx(g, i):
            pltpu.async_copy(
                idx_hbm.at[pl.ds(idx_row_base + g * _K, _K)], idx_v.at[i],
                sem_i.at[i])

        def wait_idx(i):
            pltpu.make_async_copy(
                idx_hbm.at[pl.ds(idx_row_base, _K)], idx_v.at[i], sem_i.at[i]
            ).wait()

        def fire_scatter(t, b):
            pltpu.async_copy(
                bufs.at[b], out_hbm.at[pl.ds(out_base + t * _SUBW, _SUBW)],
                sem_o.at[b])

        def drain_scatter(b):
            pltpu.make_async_copy(
                bufs.at[b], out_hbm.at[pl.ds(out_base, _SUBW)], sem_o.at[b]
            ).wait()

        cols = [
            lax.iota(jnp.int32, 16) + 16 * j for j in range(EMBED_DIM // 16)
        ]

        def fill(b, i, s):
            def mbody(m, carry):
                iv = jnp.full((16,), i, jnp.int32)
                sv = jnp.full((16,), s, jnp.int32)
                for l in range(16):
                    splat = plsc.load_gather(
                        idx_v, [iv, sv, jnp.full((16,), 16 * m + l, jnp.int32)])
                    base = splat * EMBED_DIM
                    row_off = (16 * m + l) * EMBED_DIM
                    for j in range(EMBED_DIM // 16):
                        vals = plsc.load_gather(table_tl, [base + cols[j]])
                        bufs[b, pl.ds(row_off + 16 * j, 16)] = vals
                return carry

            lax.fori_loop(0, _SUB // 16, mbody, 0)

        fire_idx(0, 0)

        def step(t, carry):
            g = t // _K
            s = t % _K
            islot = g % 2
            b = t % _NBUF

            @pl.when(s == 0)
            def _():
                wait_idx(islot)
                fire_idx(jnp.minimum(g + 1, _NGROUP - 1), 1 - islot)

            @pl.when(t >= _NBUF)
            def _():
                drain_scatter(b)

            fill(b, islot, s)
            fire_scatter(t, b)
            return carry

        lax.fori_loop(0, _NSUBW, step, 0)

        wait_idx(0)
        for b in range(_NBUF):
            drain_scatter(b)

    return emb


_emb_kernel = _make_sc_kernel()


@jax.jit
def kernel(atom_type_array, embedding_table):
    idx2d = atom_type_array.astype(jnp.int32).reshape(_B // 128, 128)
    out = _emb_kernel(embedding_table.reshape(_TW), idx2d)
    return out.reshape(atom_type_array.shape + (EMBED_DIM,))


# speedup vs baseline: 5.9898x; 5.9898x over previous
import functools

import jax
import jax.numpy as jnp
from jax import lax
from jax.experimental import pallas as pl
from jax.experimental.pallas import tpu as pltpu
from jax.experimental.pallas import tpu_sc as plsc

NUM_ELEMENTS = 100
EMBED_DIM = 128

_B = 16384 * 200
_NC = 2
_NS = 16
_NW = _NC * _NS
_BPW = _B // _NW
_K = 8
_SUB = 128
_NBUF = 6
_LAG = 2
_NGROUP = _BPW // (_K * _SUB)
_IDX_ROWS_PER_W = _BPW // 128
_NSUBW = _BPW // _SUB


def _make_sc_kernel():
    mesh = plsc.VectorSubcoreMesh(core_axis_name="c", subcore_axis_name="s")

    @functools.partial(
        pl.kernel,
        mesh=mesh,
        out_type=jax.ShapeDtypeStruct((_B, EMBED_DIM), jnp.float32),
        scratch_types=(
            [pltpu.VMEM((_K, 128), jnp.int32)] * 3
            + [pltpu.VMEM((_SUB, EMBED_DIM), jnp.float32)] * _NBUF
            + [pltpu.VMEM_SHARED((NUM_ELEMENTS, EMBED_DIM), jnp.float32)]
            + [pltpu.SemaphoreType.DMA] * (3 + 2 * _NBUF)
        ),
    )
    def emb(table_hbm, idx_hbm, out_hbm, idx0, idx1, idx2,
            buf0, buf1, buf2, buf3, buf4, buf5, table_sp,
            sem_i0, sem_i1, sem_i2,
            sem_g0, sem_g1, sem_g2, sem_g3, sem_g4, sem_g5,
            sem_o0, sem_o1, sem_o2, sem_o3, sem_o4, sem_o5):
        idxs = [idx0, idx1, idx2]
        sem_i = [sem_i0, sem_i1, sem_i2]
        bufs = [buf0, buf1, buf2, buf3, buf4, buf5]
        sem_g = [sem_g0, sem_g1, sem_g2, sem_g3, sem_g4, sem_g5]
        sem_o = [sem_o0, sem_o1, sem_o2, sem_o3, sem_o4, sem_o5]

        sid = lax.axis_index("s")
        wid = sid * _NC + lax.axis_index("c")
        idx_row_base = wid * _IDX_ROWS_PER_W
        out_base = wid * _BPW

        @pl.when(sid == 0)
        def _():
            pltpu.sync_copy(table_hbm, table_sp)

        plsc.subcore_barrier()

        def fire_idx(g, i):
            pltpu.async_copy(
                idx_hbm.at[pl.ds(idx_row_base + g * _K, _K)], idxs[i],
                sem_i[i])

        def wait_idx(i):
            pltpu.make_async_copy(
                idx_hbm.at[pl.ds(idx_row_base, _K)], idxs[i], sem_i[i]
            ).wait()

        def fire_gather(r, b, i):
            pltpu.async_copy(table_sp.at[idxs[i].at[r]], bufs[b], sem_g[b])

        def drain_gather(b):
            pltpu.make_async_copy(
                out_hbm.at[pl.ds(out_base, _SUB)], bufs[b], sem_g[b]
            ).wait()

        def fire_scatter(t, b):
            pltpu.async_copy(
                bufs[b], out_hbm.at[pl.ds(out_base + t * _SUB, _SUB)],
                sem_o[b])

        def drain_scatter(b):
            pltpu.make_async_copy(
                bufs[b], out_hbm.at[pl.ds(out_base, _SUB)], sem_o[b]
            ).wait()

        def one_group(g, r, first=False):
            wait_idx(r)
            fire_idx(jnp.minimum(g + 1, _NGROUP - 1), (r + 1) % 3)
            for s in range(_K):
                b = (2 * r + s) % _NBUF
                if not (first and s < _NBUF):
                    drain_scatter(b)
                fire_gather(s, b, r)
                if not (first and s < _LAG):
                    bp = (2 * r + s - _LAG) % _NBUF
                    drain_gather(bp)
                    fire_scatter(g * _K + s - _LAG, bp)

        fire_idx(0, 0)
        one_group(0, 0, first=True)

        def body(k, carry):
            g = 1 + 3 * k
            one_group(g, 1)
            one_group(g + 1, 2)
            one_group(g + 2, 0)
            return carry

        lax.fori_loop(0, (_NGROUP - 1) // 3, body, 0)

        for d in range(_LAG):
            t = _NSUBW - _LAG + d
            b = (t + 2 * 0) % _NBUF
            drain_gather(b)
            fire_scatter(t, b)
        wait_idx(1)
        for b in range(_NBUF):
            drain_scatter(b)

    return emb


_emb_kernel = _make_sc_kernel()


@jax.jit
def kernel(atom_type_array, embedding_table):
    idx2d = atom_type_array.astype(jnp.int32).reshape(_B // 128, 128)
    out = _emb_kernel(embedding_table, idx2d)
    return out.reshape(atom_type_array.shape + (EMBED_DIM,))
